# baseline (speedup 1.0000x reference)
import jax
import jax.numpy as jnp
from jax.experimental import pallas as pl

N = 10000
E = 320000
NUM_GRAPHS = 128
DIM = 32
NUM_CLASSES = 10


def _gat_conv(x, src, dst, W, att_src, att_dst, b):
    h = x @ W
    es = (h * att_src).sum(axis=1)
    ed = (h * att_dst).sum(axis=1)
    e = jax.nn.leaky_relu(es[src] + ed[dst], negative_slope=0.2)
    m = jax.ops.segment_max(e, dst, num_segments=N)
    m = jnp.where(jnp.isneginf(m), 0.0, m)
    ex = jnp.exp(e - m[dst])
    s = jax.ops.segment_sum(ex, dst, num_segments=N)
    alpha = ex / (s[dst] + 1e-16)
    return jax.ops.segment_sum(h[src] * alpha[:, None], dst, num_segments=N) + b


def _l2norm(x):
    n = jnp.linalg.norm(x, ord=2, axis=1, keepdims=True)
    return x / jnp.maximum(n, 1e-12)


def _head_kernel(g_ref, w1_ref, b1_ref, w2_ref, b2_ref, o_ref):
    g = g_ref[...]
    z = jax.nn.relu(g @ w1_ref[...] + b1_ref[...])
    z = z @ w2_ref[...] + b2_ref[...]
    o_ref[...] = jax.nn.log_softmax(z, axis=-1)


def kernel(x, edge_index, batch, W1, att_src1, att_dst1, b1, W2, att_src2, att_dst2, b2, W3, att_src3, att_dst3, b3, fc1_W, fc1_b, fc2_W, fc2_b):
    loop = jnp.arange(N, dtype=edge_index.dtype)
    src = jnp.concatenate([edge_index[0], loop])
    dst = jnp.concatenate([edge_index[1], loop])
    h = jax.nn.relu(_l2norm(_gat_conv(x, src, dst, W1, att_src1, att_dst1, b1)))
    h = jax.nn.relu(_l2norm(_gat_conv(h, src, dst, W2, att_src2, att_dst2, b2)))
    h = jax.nn.relu(_l2norm(_gat_conv(h, src, dst, W3, att_src3, att_dst3, b3)))
    g = jax.ops.segment_sum(h, batch, num_segments=NUM_GRAPHS)
    out = pl.pallas_call(
        _head_kernel,
        out_shape=jax.ShapeDtypeStruct((NUM_GRAPHS, NUM_CLASSES), jnp.float32),
    )(g, fc1_W, fc1_b, fc2_W, fc2_b)
    return out


# speedup vs baseline: 23.4887x; 23.4887x over previous
import dataclasses
import functools

import jax
import jax.numpy as jnp
from jax import lax
from jax.experimental import pallas as pl
from jax.experimental.pallas import tpu as pltpu
from jax.experimental.pallas import tpu_sc as plsc

N = 10000
E = 320000
F_IN = 128
DIM = 32
NUM_CLASSES = 10
NUM_GRAPHS = 128

E_TOT = E + N
NC, NS, LANES = 2, 16, 16
NW = NC * NS
EB = 128
NBLK = -(-E_TOT // (NW * EB))
E_PAD = NW * NBLK * EB
N_PAD = 10240
ROWS_PER_TEC = N_PAD // NS
UW = 48

_mesh = plsc.VectorSubcoreMesh(core_axis_name="c", subcore_axis_name="s")

_cp = pltpu.CompilerParams(needs_layout_passes=False, use_tc_tiling_on_sc=False)


def _sc_edge_body(src_hbm, dst_hbm, es_hbm, ed_hbm, h_hbm, z_hbm, u_hbm,
                  src_v, dst_v, es_v, ed_v, gin, gout, u_sh):
    cid = lax.axis_index("c")
    tid = lax.axis_index("s")
    wid = tid * NC + cid
    pltpu.sync_copy(src_hbm.at[wid], src_v)
    pltpu.sync_copy(dst_hbm.at[wid], dst_v)
    pltpu.sync_copy(es_hbm, es_v)
    pltpu.sync_copy(ed_hbm, ed_v)
    pltpu.sync_copy(z_hbm, u_sh.at[pl.ds(tid * ROWS_PER_TEC, ROWS_PER_TEC)])
    plsc.subcore_barrier()

    @pl.loop(0, NBLK)
    def _blk(b):
        pltpu.sync_copy(h_hbm.at[src_v.at[b]], gin)
        base = (wid * NBLK + b) * EB
        for j in range(EB // LANES):
            s16 = src_v[b, pl.ds(j * LANES, LANES)]
            d16 = dst_v[b, 0, pl.ds(j * LANES, LANES)]
            a = plsc.load_gather(es_v, [s16])
            dd = plsc.load_gather(ed_v, [d16])
            t = a + dd
            w = jnp.exp(jnp.maximum(t, 0.2 * t))
            gid = lax.iota(jnp.int32, 16) + (base + j * LANES)
            w = jnp.where(gid < E_TOT, w, 0.0)
            lrow = lax.iota(jnp.int32, 16) + (j * LANES)
            for f in range(DIM):
                fv = jnp.full((16,), f, jnp.int32)
                c = plsc.load_gather(gin, [lrow, fv])
                plsc.store_scatter(gout, [lrow, fv], c * w)
            for f in range(DIM, UW):
                fv = jnp.full((16,), f, jnp.int32)
                plsc.store_scatter(gout, [lrow, fv], w)
        pltpu.sync_copy(gout, u_sh.at[dst_v.at[b, 0]], add=True)

    plsc.subcore_barrier()
    pltpu.sync_copy(u_sh.at[pl.ds(tid * ROWS_PER_TEC, ROWS_PER_TEC)],
                    u_hbm.at[cid, pl.ds(tid * ROWS_PER_TEC, ROWS_PER_TEC)])


@jax.jit
def _sc_edge(src_p, dst_p, es, ed, h, zrows):
    k = pl.kernel(
        _sc_edge_body,
        out_type=jax.ShapeDtypeStruct((NC, N_PAD, UW), jnp.float32),
        mesh=_mesh,
        compiler_params=_cp,
        scratch_types=[
            pltpu.VMEM((NBLK, EB), jnp.int32),
            pltpu.VMEM((NBLK, 1, EB), jnp.int32),
            pltpu.VMEM((N,), jnp.float32),
            pltpu.VMEM((N,), jnp.float32),
            pltpu.VMEM((EB, DIM), jnp.float32),
            pltpu.VMEM((EB, UW), jnp.float32),
            pltpu.VMEM_SHARED((N_PAD, UW), jnp.float32),
        ],
    )
    return k(src_p, dst_p, es, ed, h, zrows)


def _gat_layer(h, src_p, dst_p, zrows, W, att_src, att_dst, b):
    hw = h @ W
    es = hw @ att_src
    ed = hw @ att_dst
    u = _sc_edge(src_p, dst_p, es, ed, hw, zrows)
    U = u[0, :N] + u[1, :N]
    return U[:, :DIM] / (U[:, DIM:DIM + 1] + 1e-16) + b


def _l2norm(x):
    n = jnp.linalg.norm(x, ord=2, axis=1, keepdims=True)
    return x / jnp.maximum(n, 1e-12)


def _head_kernel(g_ref, w1_ref, b1_ref, w2_ref, b2_ref, o_ref):
    g = g_ref[...]
    z = jax.nn.relu(g @ w1_ref[...] + b1_ref[...])
    z = z @ w2_ref[...] + b2_ref[...]
    o_ref[...] = jax.nn.log_softmax(z, axis=-1)


def kernel(x, edge_index, batch, W1, att_src1, att_dst1, b1, W2, att_src2, att_dst2, b2, W3, att_src3, att_dst3, b3, fc1_W, fc1_b, fc2_W, fc2_b):
    loop = jnp.arange(N, dtype=edge_index.dtype)
    src = jnp.concatenate([edge_index[0], loop])
    dst = jnp.concatenate([edge_index[1], loop])
    pad = E_PAD - E_TOT
    src_p = jnp.pad(src, (0, pad)).reshape(NW, NBLK, EB)
    dst_p = jnp.pad(dst, (0, pad)).reshape(NW, NBLK, 1, EB)
    zrows = jnp.zeros((ROWS_PER_TEC, UW), jnp.float32)

    h = jax.nn.relu(_l2norm(_gat_layer(x, src_p, dst_p, zrows, W1, att_src1, att_dst1, b1)))
    h = jax.nn.relu(_l2norm(_gat_layer(h, src_p, dst_p, zrows, W2, att_src2, att_dst2, b2)))
    h = jax.nn.relu(_l2norm(_gat_layer(h, src_p, dst_p, zrows, W3, att_src3, att_dst3, b3)))
    g = jax.ops.segment_sum(h, batch, num_segments=NUM_GRAPHS)
    out = pl.pallas_call(
        _head_kernel,
        out_shape=jax.ShapeDtypeStruct((NUM_GRAPHS, NUM_CLASSES), jnp.float32),
    )(g, fc1_W, fc1_b, fc2_W, fc2_b)
    return out


# speedup vs baseline: 57.8743x; 2.4639x over previous
import dataclasses
import functools

import jax
import jax.numpy as jnp
from jax import lax
from jax.experimental import pallas as pl
from jax.experimental.pallas import tpu as pltpu
from jax.experimental.pallas import tpu_sc as plsc

N = 10000
E = 320000
F_IN = 128
DIM = 32
NUM_CLASSES = 10
NUM_GRAPHS = 128

E_TOT = E + N
NC, NS, LANES = 2, 16, 16
NW = NC * NS
EB = 128
NBLK = 82
E_PAD = NW * NBLK * EB
N_PAD = 10240
ROWS_PER_TEC = N_PAD // NS
UW = 48

_mesh = plsc.VectorSubcoreMesh(core_axis_name="c", subcore_axis_name="s")

_cp = pltpu.CompilerParams(needs_layout_passes=False, use_tc_tiling_on_sc=False)


def _sc_edge_body(src_hbm, dst_hbm, es_hbm, ed_hbm, h_hbm, z_hbm, u_hbm,
                  src_v, dst_v, es_v, ed_v, gin0, gin1, gout0, gout1, wbuf,
                  u_sh, sem_a, sem_g0, sem_g1, sem_s0, sem_s1):
    cid = lax.axis_index("c")
    tid = lax.axis_index("s")
    wid = tid * NC + cid
    pltpu.async_copy(src_hbm.at[wid], src_v, sem_a)
    pltpu.async_copy(dst_hbm.at[wid], dst_v, sem_a)
    pltpu.async_copy(es_hbm, es_v, sem_a)
    pltpu.async_copy(ed_hbm, ed_v, sem_a)
    uslice = u_sh.at[pl.ds(tid * ROWS_PER_TEC, ROWS_PER_TEC)]
    pltpu.async_copy(z_hbm, uslice, sem_a)
    pltpu.make_async_copy(src_hbm.at[wid], src_v, sem_a).wait()
    pltpu.make_async_copy(dst_hbm.at[wid], dst_v, sem_a).wait()
    pltpu.make_async_copy(es_hbm, es_v, sem_a).wait()
    pltpu.make_async_copy(ed_hbm, ed_v, sem_a).wait()
    pltpu.make_async_copy(z_hbm, uslice, sem_a).wait()
    pltpu.async_copy(h_hbm.at[src_v.at[0]], gin0, sem_g0)
    plsc.subcore_barrier()

    def compute_block(b, gin, gout):
        base = (wid * NBLK + b) * EB
        for j in range(EB // LANES):
            s16 = src_v[b, pl.ds(j * LANES, LANES)]
            d16 = dst_v[b, 0, pl.ds(j * LANES, LANES)]
            a = plsc.load_gather(es_v, [s16])
            dd = plsc.load_gather(ed_v, [d16])
            t = a + dd
            w = jnp.exp(jnp.maximum(t, 0.2 * t))
            gid = lax.iota(jnp.int32, 16) + (base + j * LANES)
            w = jnp.where(gid < E_TOT, w, 0.0)
            for l in range(LANES):
                e = j * LANES + l
                ws = jnp.full((LANES,), w[l])
                gout[e, pl.ds(0, LANES)] = gin[e, pl.ds(0, LANES)] * ws
                gout[e, pl.ds(LANES, LANES)] = gin[e, pl.ds(LANES, LANES)] * ws
                gout[e, pl.ds(2 * LANES, LANES)] = ws

    @pl.loop(0, NBLK // 2)
    def _iter(i):
        b0 = 2 * i
        b1 = 2 * i + 1
        pltpu.async_copy(h_hbm.at[src_v.at[b1]], gin1, sem_g1)
        pltpu.make_async_copy(h_hbm.at[src_v.at[b0]], gin0, sem_g0).wait()

        @pl.when(i > 0)
        def _():
            pltpu.make_async_copy(
                gout0, u_sh.at[dst_v.at[b0 - 2, 0]], sem_s0).wait()
        compute_block(b0, gin0, gout0)
        pltpu.async_copy(gout0, u_sh.at[dst_v.at[b0, 0]], sem_s0, add=True)

        @pl.when(i < NBLK // 2 - 1)
        def _():
            pltpu.async_copy(h_hbm.at[src_v.at[b0 + 2]], gin0, sem_g0)
        pltpu.make_async_copy(h_hbm.at[src_v.at[b1]], gin1, sem_g1).wait()

        @pl.when(i > 0)
        def _():
            pltpu.make_async_copy(
                gout1, u_sh.at[dst_v.at[b1 - 2, 0]], sem_s1).wait()
        compute_block(b1, gin1, gout1)
        pltpu.async_copy(gout1, u_sh.at[dst_v.at[b1, 0]], sem_s1, add=True)

    pltpu.make_async_copy(
        gout0, u_sh.at[dst_v.at[NBLK - 2, 0]], sem_s0).wait()
    pltpu.make_async_copy(
        gout1, u_sh.at[dst_v.at[NBLK - 1, 0]], sem_s1).wait()
    plsc.subcore_barrier()
    pltpu.sync_copy(uslice,
                    u_hbm.at[cid, pl.ds(tid * ROWS_PER_TEC, ROWS_PER_TEC)])


@jax.jit
def _sc_edge(src_p, dst_p, es, ed, h, zrows):
    k = pl.kernel(
        _sc_edge_body,
        out_type=jax.ShapeDtypeStruct((NC, N_PAD, UW), jnp.float32),
        mesh=_mesh,
        compiler_params=_cp,
        scratch_types=[
            pltpu.VMEM((NBLK, EB), jnp.int32),
            pltpu.VMEM((NBLK, 1, EB), jnp.int32),
            pltpu.VMEM((N,), jnp.float32),
            pltpu.VMEM((N,), jnp.float32),
            pltpu.VMEM((EB, DIM), jnp.float32),
            pltpu.VMEM((EB, DIM), jnp.float32),
            pltpu.VMEM((EB, UW), jnp.float32),
            pltpu.VMEM((EB, UW), jnp.float32),
            pltpu.VMEM((EB,), jnp.float32),
            pltpu.VMEM_SHARED((N_PAD, UW), jnp.float32),
            pltpu.SemaphoreType.DMA,
            pltpu.SemaphoreType.DMA,
            pltpu.SemaphoreType.DMA,
            pltpu.SemaphoreType.DMA,
            pltpu.SemaphoreType.DMA,
        ],
    )
    return k(src_p, dst_p, es, ed, h, zrows)


def _gat_layer(h, src_p, dst_p, zrows, W, att_src, att_dst, b):
    hw = h @ W
    es = hw @ att_src
    ed = hw @ att_dst
    u = _sc_edge(src_p, dst_p, es, ed, hw, zrows)
    U = u[0, :N] + u[1, :N]
    return U[:, :DIM] / (U[:, DIM:DIM + 1] + 1e-16) + b


def _l2norm(x):
    n = jnp.linalg.norm(x, ord=2, axis=1, keepdims=True)
    return x / jnp.maximum(n, 1e-12)


def _head_kernel(g_ref, w1_ref, b1_ref, w2_ref, b2_ref, o_ref):
    g = g_ref[...]
    z = jax.nn.relu(g @ w1_ref[...] + b1_ref[...])
    z = z @ w2_ref[...] + b2_ref[...]
    o_ref[...] = jax.nn.log_softmax(z, axis=-1)


def kernel(x, edge_index, batch, W1, att_src1, att_dst1, b1, W2, att_src2, att_dst2, b2, W3, att_src3, att_dst3, b3, fc1_W, fc1_b, fc2_W, fc2_b):
    loop = jnp.arange(N, dtype=edge_index.dtype)
    src = jnp.concatenate([edge_index[0], loop])
    dst = jnp.concatenate([edge_index[1], loop])
    pad = E_PAD - E_TOT
    src_p = jnp.pad(src, (0, pad)).reshape(NW, NBLK, EB)
    dst_p = jnp.pad(dst, (0, pad)).reshape(NW, NBLK, 1, EB)
    zrows = jnp.zeros((ROWS_PER_TEC, UW), jnp.float32)

    h = jax.nn.relu(_l2norm(_gat_layer(x, src_p, dst_p, zrows, W1, att_src1, att_dst1, b1)))
    h = jax.nn.relu(_l2norm(_gat_layer(h, src_p, dst_p, zrows, W2, att_src2, att_dst2, b2)))
    h = jax.nn.relu(_l2norm(_gat_layer(h, src_p, dst_p, zrows, W3, att_src3, att_dst3, b3)))
    g = jax.ops.segment_sum(h, batch, num_segments=NUM_GRAPHS)
    out = pl.pallas_call(
        _head_kernel,
        out_shape=jax.ShapeDtypeStruct((NUM_GRAPHS, NUM_CLASSES), jnp.float32),
    )(g, fc1_W, fc1_b, fc2_W, fc2_b)
    return out


# speedup vs baseline: 77.8884x; 1.3458x over previous
import dataclasses
import functools

import jax
import jax.numpy as jnp
from jax import lax
from jax.experimental import pallas as pl
from jax.experimental.pallas import tpu as pltpu
from jax.experimental.pallas import tpu_sc as plsc

N = 10000
E = 320000
F_IN = 128
DIM = 32
NUM_CLASSES = 10
NUM_GRAPHS = 128

E_TOT = E + N
NC, NS, LANES = 2, 16, 16
NW = NC * NS
EB = 128
NBLK = 82
E_PAD = NW * NBLK * EB
N_PAD = 10240
ROWS_PER_TEC = N_PAD // NS
UW = 48

_mesh = plsc.VectorSubcoreMesh(core_axis_name="c", subcore_axis_name="s")

_cp = pltpu.CompilerParams(needs_layout_passes=False, use_tc_tiling_on_sc=False)


def _sc_edge_body(src_hbm, dst_hbm, es_hbm, ed_hbm, h_hbm, z_hbm, u_hbm,
                  src_v, dst_v, es_v, ed_v, gin0, gin1, gout0, gout1, wbuf,
                  u_sh, h_sh, sem_a, sem_g0, sem_g1, sem_s0, sem_s1):
    cid = lax.axis_index("c")
    tid = lax.axis_index("s")
    wid = tid * NC + cid
    pltpu.async_copy(src_hbm.at[wid], src_v, sem_a)
    pltpu.async_copy(dst_hbm.at[wid], dst_v, sem_a)
    pltpu.async_copy(es_hbm, es_v, sem_a)
    pltpu.async_copy(ed_hbm, ed_v, sem_a)
    uslice = u_sh.at[pl.ds(tid * ROWS_PER_TEC, ROWS_PER_TEC)]
    pltpu.async_copy(z_hbm, uslice, sem_a)
    hslice_hbm = h_hbm.at[pl.ds(tid * (N // NS), N // NS)]
    hslice_sh = h_sh.at[pl.ds(tid * (N // NS), N // NS)]
    pltpu.async_copy(hslice_hbm, hslice_sh, sem_a)
    pltpu.make_async_copy(src_hbm.at[wid], src_v, sem_a).wait()
    pltpu.make_async_copy(dst_hbm.at[wid], dst_v, sem_a).wait()
    pltpu.make_async_copy(es_hbm, es_v, sem_a).wait()
    pltpu.make_async_copy(ed_hbm, ed_v, sem_a).wait()
    pltpu.make_async_copy(z_hbm, uslice, sem_a).wait()
    pltpu.make_async_copy(hslice_hbm, hslice_sh, sem_a).wait()
    plsc.subcore_barrier()
    pltpu.async_copy(h_sh.at[src_v.at[0]], gin0, sem_g0)

    def compute_block(b, gin, gout):
        base = (wid * NBLK + b) * EB
        for j in range(EB // LANES):
            s16 = src_v[b, pl.ds(j * LANES, LANES)]
            d16 = dst_v[b, 0, pl.ds(j * LANES, LANES)]
            a = plsc.load_gather(es_v, [s16])
            dd = plsc.load_gather(ed_v, [d16])
            t = a + dd
            w = jnp.exp(jnp.maximum(t, 0.2 * t))
            gid = lax.iota(jnp.int32, 16) + (base + j * LANES)
            w = jnp.where(gid < E_TOT, w, 0.0)
            for l in range(LANES):
                e = j * LANES + l
                ws = jnp.full((LANES,), w[l])
                gout[e, pl.ds(0, LANES)] = gin[e, pl.ds(0, LANES)] * ws
                gout[e, pl.ds(LANES, LANES)] = gin[e, pl.ds(LANES, LANES)] * ws
                gout[e, pl.ds(2 * LANES, LANES)] = ws

    @pl.loop(0, NBLK // 2)
    def _iter(i):
        b0 = 2 * i
        b1 = 2 * i + 1
        pltpu.async_copy(h_sh.at[src_v.at[b1]], gin1, sem_g1)
        pltpu.make_async_copy(h_sh.at[src_v.at[b0]], gin0, sem_g0).wait()

        @pl.when(i > 0)
        def _():
            pltpu.make_async_copy(
                gout0, u_sh.at[dst_v.at[b0 - 2, 0]], sem_s0).wait()
        compute_block(b0, gin0, gout0)
        pltpu.async_copy(gout0, u_sh.at[dst_v.at[b0, 0]], sem_s0, add=True)

        @pl.when(i < NBLK // 2 - 1)
        def _():
            pltpu.async_copy(h_sh.at[src_v.at[b0 + 2]], gin0, sem_g0)
        pltpu.make_async_copy(h_sh.at[src_v.at[b1]], gin1, sem_g1).wait()

        @pl.when(i > 0)
        def _():
            pltpu.make_async_copy(
                gout1, u_sh.at[dst_v.at[b1 - 2, 0]], sem_s1).wait()
        compute_block(b1, gin1, gout1)
        pltpu.async_copy(gout1, u_sh.at[dst_v.at[b1, 0]], sem_s1, add=True)

    pltpu.make_async_copy(
        gout0, u_sh.at[dst_v.at[NBLK - 2, 0]], sem_s0).wait()
    pltpu.make_async_copy(
        gout1, u_sh.at[dst_v.at[NBLK - 1, 0]], sem_s1).wait()
    plsc.subcore_barrier()
    pltpu.sync_copy(uslice,
                    u_hbm.at[cid, pl.ds(tid * ROWS_PER_TEC, ROWS_PER_TEC)])


@jax.jit
def _sc_edge(src_p, dst_p, es, ed, h, zrows):
    k = pl.kernel(
        _sc_edge_body,
        out_type=jax.ShapeDtypeStruct((NC, N_PAD, UW), jnp.float32),
        mesh=_mesh,
        compiler_params=_cp,
        scratch_types=[
            pltpu.VMEM((NBLK, EB), jnp.int32),
            pltpu.VMEM((NBLK, 1, EB), jnp.int32),
            pltpu.VMEM((N,), jnp.float32),
            pltpu.VMEM((N,), jnp.float32),
            pltpu.VMEM((EB, DIM), jnp.float32),
            pltpu.VMEM((EB, DIM), jnp.float32),
            pltpu.VMEM((EB, UW), jnp.float32),
            pltpu.VMEM((EB, UW), jnp.float32),
            pltpu.VMEM((EB,), jnp.float32),
            pltpu.VMEM_SHARED((N_PAD, UW), jnp.float32),
            pltpu.VMEM_SHARED((N, DIM), jnp.float32),
            pltpu.SemaphoreType.DMA,
            pltpu.SemaphoreType.DMA,
            pltpu.SemaphoreType.DMA,
            pltpu.SemaphoreType.DMA,
            pltpu.SemaphoreType.DMA,
        ],
    )
    return k(src_p, dst_p, es, ed, h, zrows)


def _gat_layer(h, src_p, dst_p, zrows, W, att_src, att_dst, b):
    hw = h @ W
    es = hw @ att_src
    ed = hw @ att_dst
    u = _sc_edge(src_p, dst_p, es, ed, hw, zrows)
    U = u[0, :N] + u[1, :N]
    return U[:, :DIM] / (U[:, DIM:DIM + 1] + 1e-16) + b


def _l2norm(x):
    n = jnp.linalg.norm(x, ord=2, axis=1, keepdims=True)
    return x / jnp.maximum(n, 1e-12)


def _head_kernel(g_ref, w1_ref, b1_ref, w2_ref, b2_ref, o_ref):
    g = g_ref[...]
    z = jax.nn.relu(g @ w1_ref[...] + b1_ref[...])
    z = z @ w2_ref[...] + b2_ref[...]
    o_ref[...] = jax.nn.log_softmax(z, axis=-1)


def kernel(x, edge_index, batch, W1, att_src1, att_dst1, b1, W2, att_src2, att_dst2, b2, W3, att_src3, att_dst3, b3, fc1_W, fc1_b, fc2_W, fc2_b):
    loop = jnp.arange(N, dtype=edge_index.dtype)
    src = jnp.concatenate([edge_index[0], loop])
    dst = jnp.concatenate([edge_index[1], loop])
    pad = E_PAD - E_TOT
    src_p = jnp.pad(src, (0, pad)).reshape(NW, NBLK, EB)
    dst_p = jnp.pad(dst, (0, pad)).reshape(NW, NBLK, 1, EB)
    zrows = jnp.zeros((ROWS_PER_TEC, UW), jnp.float32)

    h = jax.nn.relu(_l2norm(_gat_layer(x, src_p, dst_p, zrows, W1, att_src1, att_dst1, b1)))
    h = jax.nn.relu(_l2norm(_gat_layer(h, src_p, dst_p, zrows, W2, att_src2, att_dst2, b2)))
    h = jax.nn.relu(_l2norm(_gat_layer(h, src_p, dst_p, zrows, W3, att_src3, att_dst3, b3)))
    g = jax.ops.segment_sum(h, batch, num_segments=NUM_GRAPHS)
    out = pl.pallas_call(
        _head_kernel,
        out_shape=jax.ShapeDtypeStruct((NUM_GRAPHS, NUM_CLASSES), jnp.float32),
    )(g, fc1_W, fc1_b, fc2_W, fc2_b)
    return out


# speedup vs baseline: 82.9286x; 1.0647x over previous
import jax
import jax.numpy as jnp
from jax import lax
from jax.experimental import pallas as pl
from jax.experimental.pallas import tpu as pltpu
from jax.experimental.pallas import tpu_sc as plsc

N = 10000
E = 320000
F_IN = 128
DIM = 32
NUM_CLASSES = 10
NUM_GRAPHS = 128

E_TOT = E + N
NC, NS, LANES = 2, 16, 16
NW = NC * NS
EB = 128
NBLK = 82
E_PAD = NW * NBLK * EB
N_PAD = 10240
ROWS_PER_TEC = N_PAD // NS
UW = 48
BR = 1024

_mesh = plsc.VectorSubcoreMesh(core_axis_name="c", subcore_axis_name="s")

_cp = pltpu.CompilerParams(needs_layout_passes=False, use_tc_tiling_on_sc=False)


def _sc_edge_body(src_hbm, dst_hbm, es_hbm, ed_hbm, h_hbm, z_hbm, u_hbm,
                  src_v, dst_v, es_v, ed_v, gin0, gin1, gout0, gout1,
                  u_sh, h_sh, sem_a, sem_g0, sem_g1, sem_s0, sem_s1):
    cid = lax.axis_index("c")
    tid = lax.axis_index("s")
    wid = tid * NC + cid
    pltpu.async_copy(src_hbm.at[wid], src_v, sem_a)
    pltpu.async_copy(dst_hbm.at[wid], dst_v, sem_a)
    pltpu.async_copy(es_hbm, es_v, sem_a)
    pltpu.async_copy(ed_hbm, ed_v, sem_a)
    uslice = u_sh.at[pl.ds(tid * ROWS_PER_TEC, ROWS_PER_TEC)]
    pltpu.async_copy(z_hbm, uslice, sem_a)
    hslice_hbm = h_hbm.at[pl.ds(tid * ROWS_PER_TEC, ROWS_PER_TEC)]
    hslice_sh = h_sh.at[pl.ds(tid * ROWS_PER_TEC, ROWS_PER_TEC)]
    pltpu.async_copy(hslice_hbm, hslice_sh, sem_a)
    pltpu.make_async_copy(src_hbm.at[wid], src_v, sem_a).wait()
    pltpu.make_async_copy(dst_hbm.at[wid], dst_v, sem_a).wait()
    pltpu.make_async_copy(es_hbm, es_v, sem_a).wait()
    pltpu.make_async_copy(ed_hbm, ed_v, sem_a).wait()
    pltpu.make_async_copy(z_hbm, uslice, sem_a).wait()
    pltpu.make_async_copy(hslice_hbm, hslice_sh, sem_a).wait()
    plsc.subcore_barrier()
    pltpu.async_copy(h_sh.at[src_v.at[0]], gin0, sem_g0)

    def compute_block(b, gin, gout):
        base = (wid * NBLK + b) * EB
        for j in range(EB // LANES):
            s16 = src_v[b, pl.ds(j * LANES, LANES)]
            d16 = dst_v[b, 0, pl.ds(j * LANES, LANES)]
            a = plsc.load_gather(es_v, [s16])
            dd = plsc.load_gather(ed_v, [d16])
            t = a + dd
            w = jnp.exp(jnp.maximum(t, 0.2 * t))
            gid = lax.iota(jnp.int32, 16) + (base + j * LANES)
            w = jnp.where(gid < E_TOT, w, 0.0)
            for l in range(LANES):
                e = j * LANES + l
                ws = jnp.full((LANES,), w[l])
                gout[e, pl.ds(0, LANES)] = gin[e, pl.ds(0, LANES)] * ws
                gout[e, pl.ds(LANES, LANES)] = gin[e, pl.ds(LANES, LANES)] * ws
                gout[e, pl.ds(2 * LANES, LANES)] = ws

    @pl.loop(0, NBLK // 2)
    def _iter(i):
        b0 = 2 * i
        b1 = 2 * i + 1
        pltpu.async_copy(h_sh.at[src_v.at[b1]], gin1, sem_g1)
        pltpu.make_async_copy(h_sh.at[src_v.at[b0]], gin0, sem_g0).wait()

        @pl.when(i > 0)
        def _():
            pltpu.make_async_copy(
                gout0, u_sh.at[dst_v.at[b0 - 2, 0]], sem_s0).wait()
        compute_block(b0, gin0, gout0)
        pltpu.async_copy(gout0, u_sh.at[dst_v.at[b0, 0]], sem_s0, add=True)

        @pl.when(i < NBLK // 2 - 1)
        def _():
            pltpu.async_copy(h_sh.at[src_v.at[b0 + 2]], gin0, sem_g0)
        pltpu.make_async_copy(h_sh.at[src_v.at[b1]], gin1, sem_g1).wait()

        @pl.when(i > 0)
        def _():
            pltpu.make_async_copy(
                gout1, u_sh.at[dst_v.at[b1 - 2, 0]], sem_s1).wait()
        compute_block(b1, gin1, gout1)
        pltpu.async_copy(gout1, u_sh.at[dst_v.at[b1, 0]], sem_s1, add=True)

    pltpu.make_async_copy(
        gout0, u_sh.at[dst_v.at[NBLK - 2, 0]], sem_s0).wait()
    pltpu.make_async_copy(
        gout1, u_sh.at[dst_v.at[NBLK - 1, 0]], sem_s1).wait()
    plsc.subcore_barrier()
    pltpu.sync_copy(uslice,
                    u_hbm.at[cid, pl.ds(tid * ROWS_PER_TEC, ROWS_PER_TEC)])


def _sc_edge(src_p, dst_p, es, ed, h, zrows):
    k = pl.kernel(
        _sc_edge_body,
        out_type=jax.ShapeDtypeStruct((NC, N_PAD, UW), jnp.float32),
        mesh=_mesh,
        compiler_params=_cp,
        scratch_types=[
            pltpu.VMEM((NBLK, EB), jnp.int32),
            pltpu.VMEM((NBLK, 1, EB), jnp.int32),
            pltpu.VMEM((N_PAD,), jnp.float32),
            pltpu.VMEM((N_PAD,), jnp.float32),
            pltpu.VMEM((EB, DIM), jnp.float32),
            pltpu.VMEM((EB, DIM), jnp.float32),
            pltpu.VMEM((EB, UW), jnp.float32),
            pltpu.VMEM((EB, UW), jnp.float32),
            pltpu.VMEM_SHARED((N_PAD, UW), jnp.float32),
            pltpu.VMEM_SHARED((N_PAD, DIM), jnp.float32),
            pltpu.SemaphoreType.DMA,
            pltpu.SemaphoreType.DMA,
            pltpu.SemaphoreType.DMA,
            pltpu.SemaphoreType.DMA,
            pltpu.SemaphoreType.DMA,
        ],
    )
    return k(src_p, dst_p, es, ed, h, zrows)


def _lay1_body(x_ref, w_ref, as_ref, ad_ref, h_ref, es_ref, ed_ref):
    hw = jnp.dot(x_ref[...], w_ref[...], preferred_element_type=jnp.float32)
    h_ref[...] = hw
    es_ref[...] = hw @ as_ref[...]
    ed_ref[...] = hw @ ad_ref[...]


def _tc_layer1(x_pad, W1, att_src1, att_dst1):
    g = N_PAD // BR
    return pl.pallas_call(
        _lay1_body,
        grid=(g,),
        in_specs=[
            pl.BlockSpec((BR, F_IN), lambda i: (i, 0)),
            pl.BlockSpec((F_IN, DIM), lambda i: (0, 0)),
            pl.BlockSpec((DIM,), lambda i: (0,)),
            pl.BlockSpec((DIM,), lambda i: (0,)),
        ],
        out_specs=[
            pl.BlockSpec((BR, DIM), lambda i: (i, 0)),
            pl.BlockSpec((BR,), lambda i: (i,)),
            pl.BlockSpec((BR,), lambda i: (i,)),
        ],
        out_shape=[
            jax.ShapeDtypeStruct((N_PAD, DIM), jnp.float32),
            jax.ShapeDtypeStruct((N_PAD,), jnp.float32),
            jax.ShapeDtypeStruct((N_PAD,), jnp.float32),
        ],
    )(x_pad, W1, att_src1, att_dst1)


def _combine(u_ref, b_ref):
    U = u_ref[0] + u_ref[1]
    agg = U[:, :DIM] / (U[:, DIM:DIM + 1] + 1e-16) + b_ref[...]
    nrm = jnp.sqrt(jnp.sum(agg * agg, axis=1, keepdims=True))
    return jax.nn.relu(agg / jnp.maximum(nrm, 1e-12))


def _mid_body(u_ref, b_ref, w_ref, as_ref, ad_ref, h_ref, es_ref, ed_ref):
    hn = _combine(u_ref, b_ref)
    hw = jnp.dot(hn, w_ref[...], preferred_element_type=jnp.float32)
    h_ref[...] = hw
    es_ref[...] = hw @ as_ref[...]
    ed_ref[...] = hw @ ad_ref[...]


def _tc_mid(u, b_prev, W, att_src, att_dst):
    g = N_PAD // BR
    return pl.pallas_call(
        _mid_body,
        grid=(g,),
        in_specs=[
            pl.BlockSpec((NC, BR, UW), lambda i: (0, i, 0)),
            pl.BlockSpec((DIM,), lambda i: (0,)),
            pl.BlockSpec((DIM, DIM), lambda i: (0, 0)),
            pl.BlockSpec((DIM,), lambda i: (0,)),
            pl.BlockSpec((DIM,), lambda i: (0,)),
        ],
        out_specs=[
            pl.BlockSpec((BR, DIM), lambda i: (i, 0)),
            pl.BlockSpec((BR,), lambda i: (i,)),
            pl.BlockSpec((BR,), lambda i: (i,)),
        ],
        out_shape=[
            jax.ShapeDtypeStruct((N_PAD, DIM), jnp.float32),
            jax.ShapeDtypeStruct((N_PAD,), jnp.float32),
            jax.ShapeDtypeStruct((N_PAD,), jnp.float32),
        ],
    )(u, b_prev, W, att_src, att_dst)


def _fin_body(u_ref, b_ref, h_ref):
    h_ref[...] = _combine(u_ref, b_ref)


def _tc_final(u, b3):
    g = N_PAD // BR
    return pl.pallas_call(
        _fin_body,
        grid=(g,),
        in_specs=[
            pl.BlockSpec((NC, BR, UW), lambda i: (0, i, 0)),
            pl.BlockSpec((DIM,), lambda i: (0,)),
        ],
        out_specs=pl.BlockSpec((BR, DIM), lambda i: (i, 0)),
        out_shape=jax.ShapeDtypeStruct((N_PAD, DIM), jnp.float32),
    )(u, b3)


def _head_body(g_ref, w1_ref, b1_ref, w2_ref, b2_ref, o_ref):
    g = g_ref[...]
    z = jax.nn.relu(g @ w1_ref[...] + b1_ref[...])
    z = z @ w2_ref[...] + b2_ref[...]
    o_ref[...] = jax.nn.log_softmax(z, axis=-1)


def _tc_head(g, fc1_W, fc1_b, fc2_W, fc2_b):
    return pl.pallas_call(
        _head_body,
        out_shape=jax.ShapeDtypeStruct((NUM_GRAPHS, NUM_CLASSES), jnp.float32),
    )(g, fc1_W, fc1_b, fc2_W, fc2_b)


def kernel(x, edge_index, batch, W1, att_src1, att_dst1, b1, W2, att_src2, att_dst2, b2, W3, att_src3, att_dst3, b3, fc1_W, fc1_b, fc2_W, fc2_b):
    loop = jnp.arange(N, dtype=edge_index.dtype)
    src = jnp.concatenate([edge_index[0], loop])
    dst = jnp.concatenate([edge_index[1], loop])
    pad = E_PAD - E_TOT
    src_p = jnp.pad(src, (0, pad)).reshape(NW, NBLK, EB)
    dst_p = jnp.pad(dst, (0, pad)).reshape(NW, NBLK, 1, EB)
    zrows = jnp.zeros((ROWS_PER_TEC, UW), jnp.float32)
    x_pad = jnp.pad(x, ((0, N_PAD - N), (0, 0)))

    h, es, ed = _tc_layer1(x_pad, W1, att_src1, att_dst1)
    u = _sc_edge(src_p, dst_p, es, ed, h, zrows)
    h, es, ed = _tc_mid(u, b1, W2, att_src2, att_dst2)
    u = _sc_edge(src_p, dst_p, es, ed, h, zrows)
    h, es, ed = _tc_mid(u, b2, W3, att_src3, att_dst3)
    u = _sc_edge(src_p, dst_p, es, ed, h, zrows)
    h3 = _tc_final(u, b3)

    g = jax.ops.segment_sum(h3[:N], batch, num_segments=NUM_GRAPHS)
    return _tc_head(g, fc1_W, fc1_b, fc2_W, fc2_b)


# speedup vs baseline: 94.4754x; 1.1392x over previous
import jax
import jax.numpy as jnp
from jax import lax
from jax.experimental import pallas as pl
from jax.experimental.pallas import tpu as pltpu
from jax.experimental.pallas import tpu_sc as plsc

N = 10000
E = 320000
F_IN = 128
DIM = 32
NUM_CLASSES = 10
NUM_GRAPHS = 128

E_TOT = E + N
NC, NS, LANES = 2, 16, 16
NW = NC * NS
EB = 128
NBLK = 82
E_PAD = NW * NBLK * EB
N_PAD = 10240
ROWS_PER_TEC = N_PAD // NS
UW = 48
BR = 1024

_mesh = plsc.VectorSubcoreMesh(core_axis_name="c", subcore_axis_name="s")

_cp = pltpu.CompilerParams(needs_layout_passes=False, use_tc_tiling_on_sc=False)


def _sc_edge_body(src_hbm, dst_hbm, es_hbm, ed_hbm, h_hbm, z_hbm, u_hbm,
                  src_v, dst_v, es_v, ed_v, gin0, gin1, gout0, gout1,
                  u_sh, h_sh, sem_a, sem_g0, sem_g1, sem_s0, sem_s1):
    cid = lax.axis_index("c")
    tid = lax.axis_index("s")
    wid = tid * NC + cid
    pltpu.async_copy(src_hbm.at[wid], src_v, sem_a)
    pltpu.async_copy(dst_hbm.at[wid], dst_v, sem_a)
    pltpu.async_copy(es_hbm, es_v, sem_a)
    pltpu.async_copy(ed_hbm, ed_v, sem_a)
    uslice = u_sh.at[pl.ds(tid * ROWS_PER_TEC, ROWS_PER_TEC)]
    pltpu.async_copy(z_hbm, uslice, sem_a)
    hslice_hbm = h_hbm.at[pl.ds(tid * ROWS_PER_TEC, ROWS_PER_TEC)]
    hslice_sh = h_sh.at[pl.ds(tid * ROWS_PER_TEC, ROWS_PER_TEC)]
    pltpu.async_copy(hslice_hbm, hslice_sh, sem_a)
    pltpu.make_async_copy(src_hbm.at[wid], src_v, sem_a).wait()
    pltpu.make_async_copy(dst_hbm.at[wid], dst_v, sem_a).wait()
    pltpu.make_async_copy(es_hbm, es_v, sem_a).wait()
    pltpu.make_async_copy(ed_hbm, ed_v, sem_a).wait()
    pltpu.make_async_copy(z_hbm, uslice, sem_a).wait()
    pltpu.make_async_copy(hslice_hbm, hslice_sh, sem_a).wait()
    plsc.subcore_barrier()
    pltpu.async_copy(h_sh.at[src_v.at[0]], gin0, sem_g0)

    def compute_block(b, gin, gout):
        base = (wid * NBLK + b) * EB
        for j in range(EB // LANES):
            s16 = src_v[b, pl.ds(j * LANES, LANES)]
            d16 = dst_v[b, 0, pl.ds(j * LANES, LANES)]
            a = plsc.load_gather(es_v, [s16])
            dd = plsc.load_gather(ed_v, [d16])
            t = a + dd
            w = jnp.exp(jnp.maximum(t, 0.2 * t))
            gid = lax.iota(jnp.int32, 16) + (base + j * LANES)
            w = jnp.where(gid < E_TOT, w, 0.0)
            for l in range(LANES):
                e = j * LANES + l
                ws = jnp.full((LANES,), w[l])
                gout[e, pl.ds(0, LANES)] = gin[e, pl.ds(0, LANES)] * ws
                gout[e, pl.ds(LANES, LANES)] = gin[e, pl.ds(LANES, LANES)] * ws
                gout[e, pl.ds(2 * LANES, LANES)] = ws

    @pl.loop(0, NBLK // 2)
    def _iter(i):
        b0 = 2 * i
        b1 = 2 * i + 1
        pltpu.async_copy(h_sh.at[src_v.at[b1]], gin1, sem_g1)
        pltpu.make_async_copy(h_sh.at[src_v.at[b0]], gin0, sem_g0).wait()

        @pl.when(i > 0)
        def _():
            pltpu.make_async_copy(
                gout0, u_sh.at[dst_v.at[b0 - 2, 0]], sem_s0).wait()
        compute_block(b0, gin0, gout0)
        pltpu.async_copy(gout0, u_sh.at[dst_v.at[b0, 0]], sem_s0, add=True)

        @pl.when(i < NBLK // 2 - 1)
        def _():
            pltpu.async_copy(h_sh.at[src_v.at[b0 + 2]], gin0, sem_g0)
        pltpu.make_async_copy(h_sh.at[src_v.at[b1]], gin1, sem_g1).wait()

        @pl.when(i > 0)
        def _():
            pltpu.make_async_copy(
                gout1, u_sh.at[dst_v.at[b1 - 2, 0]], sem_s1).wait()
        compute_block(b1, gin1, gout1)
        pltpu.async_copy(gout1, u_sh.at[dst_v.at[b1, 0]], sem_s1, add=True)

    pltpu.make_async_copy(
        gout0, u_sh.at[dst_v.at[NBLK - 2, 0]], sem_s0).wait()
    pltpu.make_async_copy(
        gout1, u_sh.at[dst_v.at[NBLK - 1, 0]], sem_s1).wait()
    plsc.subcore_barrier()
    pltpu.sync_copy(uslice,
                    u_hbm.at[cid, pl.ds(tid * ROWS_PER_TEC, ROWS_PER_TEC)])


def _sc_edge(src_p, dst_p, es, ed, h, zrows):
    k = pl.kernel(
        _sc_edge_body,
        out_type=jax.ShapeDtypeStruct((NC, N_PAD, UW), jnp.float32),
        mesh=_mesh,
        compiler_params=_cp,
        scratch_types=[
            pltpu.VMEM((NBLK, EB), jnp.int32),
            pltpu.VMEM((NBLK, 1, EB), jnp.int32),
            pltpu.VMEM((N_PAD,), jnp.float32),
            pltpu.VMEM((N_PAD,), jnp.float32),
            pltpu.VMEM((EB, DIM), jnp.float32),
            pltpu.VMEM((EB, DIM), jnp.float32),
            pltpu.VMEM((EB, UW), jnp.float32),
            pltpu.VMEM((EB, UW), jnp.float32),
            pltpu.VMEM_SHARED((N_PAD, UW), jnp.float32),
            pltpu.VMEM_SHARED((N_PAD, DIM), jnp.float32),
            pltpu.SemaphoreType.DMA,
            pltpu.SemaphoreType.DMA,
            pltpu.SemaphoreType.DMA,
            pltpu.SemaphoreType.DMA,
            pltpu.SemaphoreType.DMA,
        ],
    )
    return k(src_p, dst_p, es, ed, h, zrows)


G_PAD = 256
PB = 80
NPB = 4


def _sc_pool_body(h_hbm, batch_hbm, zg_hbm, g_hbm, h_v, b_v, g_sh, sem):
    cid = lax.axis_index("c")
    tid = lax.axis_index("s")
    wid = tid * NC + cid
    rows = NPB * PB
    pltpu.async_copy(h_hbm.at[pl.ds(wid * rows, rows)], h_v, sem)
    pltpu.async_copy(batch_hbm.at[wid], b_v, sem)

    @pl.when(tid == 0)
    def _():
        pltpu.async_copy(zg_hbm, g_sh, sem)
        pltpu.make_async_copy(zg_hbm, g_sh, sem).wait()
    pltpu.make_async_copy(h_hbm.at[pl.ds(wid * rows, rows)], h_v, sem).wait()
    pltpu.make_async_copy(batch_hbm.at[wid], b_v, sem).wait()
    plsc.subcore_barrier()
    for k in range(NPB):
        pltpu.async_copy(
            h_v.at[pl.ds(k * PB, PB)], g_sh.at[b_v.at[k, 0]], sem, add=True)
    for k in range(NPB):
        pltpu.make_async_copy(
            h_v.at[pl.ds(k * PB, PB)], g_sh.at[b_v.at[k, 0]], sem).wait()
    plsc.subcore_barrier()

    @pl.when(tid == 0)
    def _():
        pltpu.sync_copy(g_sh, g_hbm.at[cid])


def _sc_pool(h3, batch_p, zg):
    k = pl.kernel(
        _sc_pool_body,
        out_type=jax.ShapeDtypeStruct((NC, G_PAD, DIM), jnp.float32),
        mesh=_mesh,
        compiler_params=_cp,
        scratch_types=[
            pltpu.VMEM((NPB * PB, DIM), jnp.float32),
            pltpu.VMEM((NPB, 1, PB), jnp.int32),
            pltpu.VMEM_SHARED((G_PAD, DIM), jnp.float32),
            pltpu.SemaphoreType.DMA,
        ],
    )
    return k(h3, batch_p, zg)


def _lay1_body(x_ref, w_ref, as_ref, ad_ref, h_ref, es_ref, ed_ref):
    hw = jnp.dot(x_ref[...], w_ref[...], preferred_element_type=jnp.float32)
    h_ref[...] = hw
    es_ref[...] = hw @ as_ref[...]
    ed_ref[...] = hw @ ad_ref[...]


def _tc_layer1(x_pad, W1, att_src1, att_dst1):
    g = N_PAD // BR
    return pl.pallas_call(
        _lay1_body,
        grid=(g,),
        in_specs=[
            pl.BlockSpec((BR, F_IN), lambda i: (i, 0)),
            pl.BlockSpec((F_IN, DIM), lambda i: (0, 0)),
            pl.BlockSpec((DIM,), lambda i: (0,)),
            pl.BlockSpec((DIM,), lambda i: (0,)),
        ],
        out_specs=[
            pl.BlockSpec((BR, DIM), lambda i: (i, 0)),
            pl.BlockSpec((BR,), lambda i: (i,)),
            pl.BlockSpec((BR,), lambda i: (i,)),
        ],
        out_shape=[
            jax.ShapeDtypeStruct((N_PAD, DIM), jnp.float32),
            jax.ShapeDtypeStruct((N_PAD,), jnp.float32),
            jax.ShapeDtypeStruct((N_PAD,), jnp.float32),
        ],
    )(x_pad, W1, att_src1, att_dst1)


def _combine(u_ref, b_ref):
    U = u_ref[0] + u_ref[1]
    agg = U[:, :DIM] / (U[:, DIM:DIM + 1] + 1e-16) + b_ref[...]
    nrm = jnp.sqrt(jnp.sum(agg * agg, axis=1, keepdims=True))
    return jax.nn.relu(agg / jnp.maximum(nrm, 1e-12))


def _mid_body(u_ref, b_ref, w_ref, as_ref, ad_ref, h_ref, es_ref, ed_ref):
    hn = _combine(u_ref, b_ref)
    hw = jnp.dot(hn, w_ref[...], preferred_element_type=jnp.float32)
    h_ref[...] = hw
    es_ref[...] = hw @ as_ref[...]
    ed_ref[...] = hw @ ad_ref[...]


def _tc_mid(u, b_prev, W, att_src, att_dst):
    g = N_PAD // BR
    return pl.pallas_call(
        _mid_body,
        grid=(g,),
        in_specs=[
            pl.BlockSpec((NC, BR, UW), lambda i: (0, i, 0)),
            pl.BlockSpec((DIM,), lambda i: (0,)),
            pl.BlockSpec((DIM, DIM), lambda i: (0, 0)),
            pl.BlockSpec((DIM,), lambda i: (0,)),
            pl.BlockSpec((DIM,), lambda i: (0,)),
        ],
        out_specs=[
            pl.BlockSpec((BR, DIM), lambda i: (i, 0)),
            pl.BlockSpec((BR,), lambda i: (i,)),
            pl.BlockSpec((BR,), lambda i: (i,)),
        ],
        out_shape=[
            jax.ShapeDtypeStruct((N_PAD, DIM), jnp.float32),
            jax.ShapeDtypeStruct((N_PAD,), jnp.float32),
            jax.ShapeDtypeStruct((N_PAD,), jnp.float32),
        ],
    )(u, b_prev, W, att_src, att_dst)


def _fin_body(u_ref, b_ref, h_ref):
    h_ref[...] = _combine(u_ref, b_ref)


def _tc_final(u, b3):
    g = N_PAD // BR
    return pl.pallas_call(
        _fin_body,
        grid=(g,),
        in_specs=[
            pl.BlockSpec((NC, BR, UW), lambda i: (0, i, 0)),
            pl.BlockSpec((DIM,), lambda i: (0,)),
        ],
        out_specs=pl.BlockSpec((BR, DIM), lambda i: (i, 0)),
        out_shape=jax.ShapeDtypeStruct((N_PAD, DIM), jnp.float32),
    )(u, b3)


def _head_body(g_ref, w1_ref, b1_ref, w2_ref, b2_ref, o_ref):
    gall = g_ref[...]
    g = gall[0, :NUM_GRAPHS] + gall[1, :NUM_GRAPHS]
    z = jax.nn.relu(g @ w1_ref[...] + b1_ref[...])
    z = z @ w2_ref[...] + b2_ref[...]
    o_ref[...] = jax.nn.log_softmax(z, axis=-1)


def _tc_head(g, fc1_W, fc1_b, fc2_W, fc2_b):
    return pl.pallas_call(
        _head_body,
        out_shape=jax.ShapeDtypeStruct((NUM_GRAPHS, NUM_CLASSES), jnp.float32),
    )(g, fc1_W, fc1_b, fc2_W, fc2_b)


def kernel(x, edge_index, batch, W1, att_src1, att_dst1, b1, W2, att_src2, att_dst2, b2, W3, att_src3, att_dst3, b3, fc1_W, fc1_b, fc2_W, fc2_b):
    loop = jnp.arange(N, dtype=edge_index.dtype)
    src = jnp.concatenate([edge_index[0], loop])
    dst = jnp.concatenate([edge_index[1], loop])
    pad = E_PAD - E_TOT
    src_p = jnp.pad(src, (0, pad)).reshape(NW, NBLK, EB)
    dst_p = jnp.pad(dst, (0, pad)).reshape(NW, NBLK, 1, EB)
    zrows = jnp.zeros((ROWS_PER_TEC, UW), jnp.float32)
    x_pad = jnp.pad(x, ((0, N_PAD - N), (0, 0)))
    batch_p = jnp.pad(batch, (0, N_PAD - N),
                      constant_values=NUM_GRAPHS).reshape(NW, NPB, 1, PB)
    zg = jnp.zeros((G_PAD, DIM), jnp.float32)

    h, es, ed = _tc_layer1(x_pad, W1, att_src1, att_dst1)
    u = _sc_edge(src_p, dst_p, es, ed, h, zrows)
    h, es, ed = _tc_mid(u, b1, W2, att_src2, att_dst2)
    u = _sc_edge(src_p, dst_p, es, ed, h, zrows)
    h, es, ed = _tc_mid(u, b2, W3, att_src3, att_dst3)
    u = _sc_edge(src_p, dst_p, es, ed, h, zrows)
    h3 = _tc_final(u, b3)

    g = _sc_pool(h3, batch_p, zg)
    return _tc_head(g, fc1_W, fc1_b, fc2_W, fc2_b)


# speedup vs baseline: 106.0556x; 1.1226x over previous
import jax
import jax.numpy as jnp
from jax import lax
from jax.experimental import pallas as pl
from jax.experimental.pallas import tpu as pltpu
from jax.experimental.pallas import tpu_sc as plsc

N = 10000
E = 320000
F_IN = 128
DIM = 32
NUM_CLASSES = 10
NUM_GRAPHS = 128

E_TOT = E + N
NC, NS, LANES = 2, 16, 16
NW = NC * NS
EB = 128
NBLK = 82
E_PAD = NW * NBLK * EB
N_PAD = 10240
ROWS_PER_TEC = N_PAD // NS
UW = 32
BR = 1024

_mesh = plsc.VectorSubcoreMesh(core_axis_name="c", subcore_axis_name="s")

_cp = pltpu.CompilerParams(needs_layout_passes=False, use_tc_tiling_on_sc=False)


def _sc_edge_body(src_hbm, dst_hbm, es_hbm, ed_hbm, h_hbm, z_hbm, z1_hbm,
                  u_hbm, s_hbm,
                  src_v, dst_v, es_v, ed_v, gin0, gin1, gout0, gout1, s_part,
                  u_sh, h_sh, sem_a, sem_g0, sem_g1, sem_s0, sem_s1):
    cid = lax.axis_index("c")
    tid = lax.axis_index("s")
    wid = tid * NC + cid
    pltpu.async_copy(src_hbm.at[wid], src_v, sem_a)
    pltpu.async_copy(dst_hbm.at[wid], dst_v, sem_a)
    pltpu.async_copy(es_hbm, es_v, sem_a)
    pltpu.async_copy(ed_hbm, ed_v, sem_a)
    uslice = u_sh.at[pl.ds(tid * ROWS_PER_TEC, ROWS_PER_TEC)]
    pltpu.async_copy(z_hbm, uslice, sem_a)
    pltpu.async_copy(z1_hbm, s_part, sem_a)
    hslice_hbm = h_hbm.at[pl.ds(tid * ROWS_PER_TEC, ROWS_PER_TEC)]
    hslice_sh = h_sh.at[pl.ds(tid * ROWS_PER_TEC, ROWS_PER_TEC)]
    pltpu.async_copy(hslice_hbm, hslice_sh, sem_a)
    pltpu.make_async_copy(src_hbm.at[wid], src_v, sem_a).wait()
    pltpu.make_async_copy(dst_hbm.at[wid], dst_v, sem_a).wait()
    pltpu.make_async_copy(es_hbm, es_v, sem_a).wait()
    pltpu.make_async_copy(ed_hbm, ed_v, sem_a).wait()
    pltpu.make_async_copy(z_hbm, uslice, sem_a).wait()
    pltpu.make_async_copy(z1_hbm, s_part, sem_a).wait()
    pltpu.make_async_copy(hslice_hbm, hslice_sh, sem_a).wait()
    plsc.subcore_barrier()
    pltpu.async_copy(h_sh.at[src_v.at[0]], gin0, sem_g0)

    def compute_block(b, gin, gout):
        base = (wid * NBLK + b) * EB
        for j in range(EB // LANES):
            s16 = src_v[b, pl.ds(j * LANES, LANES)]
            d16 = dst_v[b, 0, pl.ds(j * LANES, LANES)]
            a = plsc.load_gather(es_v, [s16])
            dd = plsc.load_gather(ed_v, [d16])
            t = a + dd
            w = jnp.exp(jnp.maximum(t, 0.2 * t))
            gid = lax.iota(jnp.int32, 16) + (base + j * LANES)
            w = jnp.where(gid < E_TOT, w, 0.0)
            plsc.addupdate_scatter(s_part, [d16], w)
            for l in range(LANES):
                e = j * LANES + l
                ws = jnp.full((LANES,), w[l])
                gout[e, pl.ds(0, LANES)] = gin[e, pl.ds(0, LANES)] * ws
                gout[e, pl.ds(LANES, LANES)] = gin[e, pl.ds(LANES, LANES)] * ws

    @pl.loop(0, NBLK // 2)
    def _iter(i):
        b0 = 2 * i
        b1 = 2 * i + 1
        pltpu.async_copy(h_sh.at[src_v.at[b1]], gin1, sem_g1)
        pltpu.make_async_copy(h_sh.at[src_v.at[b0]], gin0, sem_g0).wait()

        @pl.when(i > 0)
        def _():
            pltpu.make_async_copy(
                gout0, u_sh.at[dst_v.at[b0 - 2, 0]], sem_s0).wait()
        compute_block(b0, gin0, gout0)
        pltpu.async_copy(gout0, u_sh.at[dst_v.at[b0, 0]], sem_s0, add=True)

        @pl.when(i < NBLK // 2 - 1)
        def _():
            pltpu.async_copy(h_sh.at[src_v.at[b0 + 2]], gin0, sem_g0)
        pltpu.make_async_copy(h_sh.at[src_v.at[b1]], gin1, sem_g1).wait()

        @pl.when(i > 0)
        def _():
            pltpu.make_async_copy(
                gout1, u_sh.at[dst_v.at[b1 - 2, 0]], sem_s1).wait()
        compute_block(b1, gin1, gout1)
        pltpu.async_copy(gout1, u_sh.at[dst_v.at[b1, 0]], sem_s1, add=True)

    pltpu.make_async_copy(
        gout0, u_sh.at[dst_v.at[NBLK - 2, 0]], sem_s0).wait()
    pltpu.make_async_copy(
        gout1, u_sh.at[dst_v.at[NBLK - 1, 0]], sem_s1).wait()
    pltpu.sync_copy(s_part, s_hbm.at[wid])
    plsc.subcore_barrier()
    pltpu.sync_copy(uslice,
                    u_hbm.at[cid, pl.ds(tid * ROWS_PER_TEC, ROWS_PER_TEC)])


def _sc_edge(src_p, dst_p, es, ed, h, zrows, zs):
    k = pl.kernel(
        _sc_edge_body,
        out_type=[jax.ShapeDtypeStruct((NC, N_PAD, UW), jnp.float32),
                  jax.ShapeDtypeStruct((NW, N_PAD), jnp.float32)],
        mesh=_mesh,
        compiler_params=_cp,
        scratch_types=[
            pltpu.VMEM((NBLK, EB), jnp.int32),
            pltpu.VMEM((NBLK, 1, EB), jnp.int32),
            pltpu.VMEM((N_PAD,), jnp.float32),
            pltpu.VMEM((N_PAD,), jnp.float32),
            pltpu.VMEM((EB, DIM), jnp.float32),
            pltpu.VMEM((EB, DIM), jnp.float32),
            pltpu.VMEM((EB, UW), jnp.float32),
            pltpu.VMEM((EB, UW), jnp.float32),
            pltpu.VMEM((N_PAD,), jnp.float32),
            pltpu.VMEM_SHARED((N_PAD, UW), jnp.float32),
            pltpu.VMEM_SHARED((N_PAD, DIM), jnp.float32),
            pltpu.SemaphoreType.DMA,
            pltpu.SemaphoreType.DMA,
            pltpu.SemaphoreType.DMA,
            pltpu.SemaphoreType.DMA,
            pltpu.SemaphoreType.DMA,
        ],
    )
    return k(src_p, dst_p, es, ed, h, zrows, zs)


G_PAD = 256
PB = 80
NPB = 4


def _sc_pool_body(h_hbm, batch_hbm, zg_hbm, g_hbm, h_v, b_v, g_sh, sem):
    cid = lax.axis_index("c")
    tid = lax.axis_index("s")
    wid = tid * NC + cid
    rows = NPB * PB
    pltpu.async_copy(h_hbm.at[pl.ds(wid * rows, rows)], h_v, sem)
    pltpu.async_copy(batch_hbm.at[wid], b_v, sem)

    @pl.when(tid == 0)
    def _():
        pltpu.async_copy(zg_hbm, g_sh, sem)
        pltpu.make_async_copy(zg_hbm, g_sh, sem).wait()
    pltpu.make_async_copy(h_hbm.at[pl.ds(wid * rows, rows)], h_v, sem).wait()
    pltpu.make_async_copy(batch_hbm.at[wid], b_v, sem).wait()
    plsc.subcore_barrier()
    for k in range(NPB):
        pltpu.async_copy(
            h_v.at[pl.ds(k * PB, PB)], g_sh.at[b_v.at[k, 0]], sem, add=True)
    for k in range(NPB):
        pltpu.make_async_copy(
            h_v.at[pl.ds(k * PB, PB)], g_sh.at[b_v.at[k, 0]], sem).wait()
    plsc.subcore_barrier()

    @pl.when(tid == 0)
    def _():
        pltpu.sync_copy(g_sh, g_hbm.at[cid])


def _sc_pool(h3, batch_p, zg):
    k = pl.kernel(
        _sc_pool_body,
        out_type=jax.ShapeDtypeStruct((NC, G_PAD, DIM), jnp.float32),
        mesh=_mesh,
        compiler_params=_cp,
        scratch_types=[
            pltpu.VMEM((NPB * PB, DIM), jnp.float32),
            pltpu.VMEM((NPB, 1, PB), jnp.int32),
            pltpu.VMEM_SHARED((G_PAD, DIM), jnp.float32),
            pltpu.SemaphoreType.DMA,
        ],
    )
    return k(h3, batch_p, zg)


def _lay1_body(x_ref, w_ref, as_ref, ad_ref, h_ref, es_ref, ed_ref):
    hw = jnp.dot(x_ref[...], w_ref[...], preferred_element_type=jnp.float32)
    h_ref[...] = hw
    es_ref[...] = hw @ as_ref[...]
    ed_ref[...] = hw @ ad_ref[...]


def _tc_layer1(x_pad, W1, att_src1, att_dst1):
    g = N_PAD // BR
    return pl.pallas_call(
        _lay1_body,
        grid=(g,),
        in_specs=[
            pl.BlockSpec((BR, F_IN), lambda i: (i, 0)),
            pl.BlockSpec((F_IN, DIM), lambda i: (0, 0)),
            pl.BlockSpec((DIM,), lambda i: (0,)),
            pl.BlockSpec((DIM,), lambda i: (0,)),
        ],
        out_specs=[
            pl.BlockSpec((BR, DIM), lambda i: (i, 0)),
            pl.BlockSpec((BR,), lambda i: (i,)),
            pl.BlockSpec((BR,), lambda i: (i,)),
        ],
        out_shape=[
            jax.ShapeDtypeStruct((N_PAD, DIM), jnp.float32),
            jax.ShapeDtypeStruct((N_PAD,), jnp.float32),
            jax.ShapeDtypeStruct((N_PAD,), jnp.float32),
        ],
    )(x_pad, W1, att_src1, att_dst1)


def _combine(u_ref, s_ref, b_ref):
    U = u_ref[0] + u_ref[1]
    den = jnp.sum(s_ref[...], axis=0)[:, None]
    agg = U[:, :DIM] / (den + 1e-16) + b_ref[...]
    nrm = jnp.sqrt(jnp.sum(agg * agg, axis=1, keepdims=True))
    return jax.nn.relu(agg / jnp.maximum(nrm, 1e-12))


def _mid_body(u_ref, s_ref, b_ref, w_ref, as_ref, ad_ref, h_ref, es_ref, ed_ref):
    hn = _combine(u_ref, s_ref, b_ref)
    hw = jnp.dot(hn, w_ref[...], preferred_element_type=jnp.float32)
    h_ref[...] = hw
    es_ref[...] = hw @ as_ref[...]
    ed_ref[...] = hw @ ad_ref[...]


def _tc_mid(u, s, b_prev, W, att_src, att_dst):
    g = N_PAD // BR
    return pl.pallas_call(
        _mid_body,
        grid=(g,),
        in_specs=[
            pl.BlockSpec((NC, BR, UW), lambda i: (0, i, 0)),
            pl.BlockSpec((NW, BR), lambda i: (0, i)),
            pl.BlockSpec((DIM,), lambda i: (0,)),
            pl.BlockSpec((DIM, DIM), lambda i: (0, 0)),
            pl.BlockSpec((DIM,), lambda i: (0,)),
            pl.BlockSpec((DIM,), lambda i: (0,)),
        ],
        out_specs=[
            pl.BlockSpec((BR, DIM), lambda i: (i, 0)),
            pl.BlockSpec((BR,), lambda i: (i,)),
            pl.BlockSpec((BR,), lambda i: (i,)),
        ],
        out_shape=[
            jax.ShapeDtypeStruct((N_PAD, DIM), jnp.float32),
            jax.ShapeDtypeStruct((N_PAD,), jnp.float32),
            jax.ShapeDtypeStruct((N_PAD,), jnp.float32),
        ],
    )(u, s, b_prev, W, att_src, att_dst)


def _fin_body(u_ref, s_ref, b_ref, h_ref):
    h_ref[...] = _combine(u_ref, s_ref, b_ref)


def _tc_final(u, s, b3):
    g = N_PAD // BR
    return pl.pallas_call(
        _fin_body,
        grid=(g,),
        in_specs=[
            pl.BlockSpec((NC, BR, UW), lambda i: (0, i, 0)),
            pl.BlockSpec((NW, BR), lambda i: (0, i)),
            pl.BlockSpec((DIM,), lambda i: (0,)),
        ],
        out_specs=pl.BlockSpec((BR, DIM), lambda i: (i, 0)),
        out_shape=jax.ShapeDtypeStruct((N_PAD, DIM), jnp.float32),
    )(u, s, b3)


def _head_body(g_ref, w1_ref, b1_ref, w2_ref, b2_ref, o_ref):
    gall = g_ref[...]
    g = gall[0, :NUM_GRAPHS] + gall[1, :NUM_GRAPHS]
    z = jax.nn.relu(g @ w1_ref[...] + b1_ref[...])
    z = z @ w2_ref[...] + b2_ref[...]
    o_ref[...] = jax.nn.log_softmax(z, axis=-1)


def _tc_head(g, fc1_W, fc1_b, fc2_W, fc2_b):
    return pl.pallas_call(
        _head_body,
        out_shape=jax.ShapeDtypeStruct((NUM_GRAPHS, NUM_CLASSES), jnp.float32),
    )(g, fc1_W, fc1_b, fc2_W, fc2_b)


def kernel(x, edge_index, batch, W1, att_src1, att_dst1, b1, W2, att_src2, att_dst2, b2, W3, att_src3, att_dst3, b3, fc1_W, fc1_b, fc2_W, fc2_b):
    loop = jnp.arange(N, dtype=edge_index.dtype)
    src = jnp.concatenate([edge_index[0], loop])
    dst = jnp.concatenate([edge_index[1], loop])
    pad = E_PAD - E_TOT
    src_p = jnp.pad(src, (0, pad)).reshape(NW, NBLK, EB)
    dst_p = jnp.pad(dst, (0, pad)).reshape(NW, NBLK, 1, EB)
    zrows = jnp.zeros((ROWS_PER_TEC, UW), jnp.float32)
    zs = jnp.zeros((N_PAD,), jnp.float32)
    x_pad = jnp.pad(x, ((0, N_PAD - N), (0, 0)))
    batch_p = jnp.pad(batch, (0, N_PAD - N),
                      constant_values=NUM_GRAPHS).reshape(NW, NPB, 1, PB)
    zg = jnp.zeros((G_PAD, DIM), jnp.float32)

    h, es, ed = _tc_layer1(x_pad, W1, att_src1, att_dst1)
    u, sden = _sc_edge(src_p, dst_p, es, ed, h, zrows, zs)
    h, es, ed = _tc_mid(u, sden, b1, W2, att_src2, att_dst2)
    u, sden = _sc_edge(src_p, dst_p, es, ed, h, zrows, zs)
    h, es, ed = _tc_mid(u, sden, b2, W3, att_src3, att_dst3)
    u, sden = _sc_edge(src_p, dst_p, es, ed, h, zrows, zs)
    h3 = _tc_final(u, sden, b3)

    g = _sc_pool(h3, batch_p, zg)
    return _tc_head(g, fc1_W, fc1_b, fc2_W, fc2_b)


# speedup vs baseline: 119.0389x; 1.1224x over previous
import jax
import jax.numpy as jnp
from jax import lax
from jax.experimental import pallas as pl
from jax.experimental.pallas import tpu as pltpu
from jax.experimental.pallas import tpu_sc as plsc

N = 10000
E = 320000
F_IN = 128
DIM = 32
NUM_CLASSES = 10
NUM_GRAPHS = 128

E_TOT = E + N
NC, NS, LANES = 2, 16, 16
NW = NC * NS
EB = 128
NBLK = 82
E_PAD = NW * NBLK * EB
N_PAD = 10240
ROWS_PER_TEC = N_PAD // NS
UW = 32
BR = 1024

_mesh = plsc.VectorSubcoreMesh(core_axis_name="c", subcore_axis_name="s")

_cp = pltpu.CompilerParams(needs_layout_passes=False, use_tc_tiling_on_sc=False)


def _sc_edge_body(src_hbm, dst_hbm, es_hbm, ed_hbm, h_hbm, z_hbm, z1_hbm,
                  u_hbm, s_hbm,
                  src_v, dst_v, es_v, ed_v, gin0, gin1, gout0, gout1, s_part,
                  u_sh, h_sh, sem_a, sem_g0, sem_g1, sem_s0, sem_s1):
    cid = lax.axis_index("c")
    tid = lax.axis_index("s")
    wid = tid * NC + cid
    pltpu.async_copy(src_hbm.at[wid], src_v, sem_a)
    pltpu.async_copy(dst_hbm.at[wid], dst_v, sem_a)
    pltpu.async_copy(es_hbm, es_v, sem_a)
    pltpu.async_copy(ed_hbm, ed_v, sem_a)
    uslice = u_sh.at[pl.ds(tid * ROWS_PER_TEC, ROWS_PER_TEC)]
    pltpu.async_copy(z_hbm, uslice, sem_a)
    pltpu.async_copy(z1_hbm, s_part, sem_a)
    hslice_hbm = h_hbm.at[pl.ds(tid * ROWS_PER_TEC, ROWS_PER_TEC),
                          pl.ds(0, DIM)]
    hslice_sh = h_sh.at[pl.ds(tid * ROWS_PER_TEC, ROWS_PER_TEC)]
    pltpu.async_copy(hslice_hbm, hslice_sh, sem_a)
    pltpu.make_async_copy(src_hbm.at[wid], src_v, sem_a).wait()
    pltpu.make_async_copy(dst_hbm.at[wid], dst_v, sem_a).wait()
    pltpu.make_async_copy(es_hbm, es_v, sem_a).wait()
    pltpu.make_async_copy(ed_hbm, ed_v, sem_a).wait()
    pltpu.make_async_copy(z_hbm, uslice, sem_a).wait()
    pltpu.make_async_copy(z1_hbm, s_part, sem_a).wait()
    pltpu.make_async_copy(hslice_hbm, hslice_sh, sem_a).wait()
    plsc.subcore_barrier()
    pltpu.async_copy(h_sh.at[src_v.at[0]], gin0, sem_g0)

    def compute_block(b, gin, gout):
        base = (wid * NBLK + b) * EB
        for j in range(EB // LANES):
            s16 = src_v[b, pl.ds(j * LANES, LANES)]
            d16 = dst_v[b, 0, pl.ds(j * LANES, LANES)]
            a = plsc.load_gather(es_v, [s16])
            dd = plsc.load_gather(ed_v, [d16])
            t = a + dd
            w = jnp.exp(jnp.maximum(t, 0.2 * t))
            gid = lax.iota(jnp.int32, 16) + (base + j * LANES)
            w = jnp.where(gid < E_TOT, w, 0.0)
            plsc.addupdate_scatter(s_part, [d16], w)
            for l in range(LANES):
                e = j * LANES + l
                ws = jnp.full((LANES,), w[l])
                gout[e, pl.ds(0, LANES)] = gin[e, pl.ds(0, LANES)] * ws
                gout[e, pl.ds(LANES, LANES)] = gin[e, pl.ds(LANES, LANES)] * ws

    @pl.loop(0, NBLK // 2)
    def _iter(i):
        b0 = 2 * i
        b1 = 2 * i + 1
        pltpu.async_copy(h_sh.at[src_v.at[b1]], gin1, sem_g1)
        pltpu.make_async_copy(h_sh.at[src_v.at[b0]], gin0, sem_g0).wait()

        @pl.when(i > 0)
        def _():
            pltpu.make_async_copy(
                gout0, u_sh.at[dst_v.at[b0 - 2, 0]], sem_s0).wait()
        compute_block(b0, gin0, gout0)
        pltpu.async_copy(gout0, u_sh.at[dst_v.at[b0, 0]], sem_s0, add=True)

        @pl.when(i < NBLK // 2 - 1)
        def _():
            pltpu.async_copy(h_sh.at[src_v.at[b0 + 2]], gin0, sem_g0)
        pltpu.make_async_copy(h_sh.at[src_v.at[b1]], gin1, sem_g1).wait()

        @pl.when(i > 0)
        def _():
            pltpu.make_async_copy(
                gout1, u_sh.at[dst_v.at[b1 - 2, 0]], sem_s1).wait()
        compute_block(b1, gin1, gout1)
        pltpu.async_copy(gout1, u_sh.at[dst_v.at[b1, 0]], sem_s1, add=True)

    pltpu.make_async_copy(
        gout0, u_sh.at[dst_v.at[NBLK - 2, 0]], sem_s0).wait()
    pltpu.make_async_copy(
        gout1, u_sh.at[dst_v.at[NBLK - 1, 0]], sem_s1).wait()
    pltpu.sync_copy(s_part, s_hbm.at[wid])
    plsc.subcore_barrier()
    pltpu.sync_copy(uslice,
                    u_hbm.at[cid, pl.ds(tid * ROWS_PER_TEC, ROWS_PER_TEC),
                             pl.ds(0, DIM)])


def _sc_edge(src_p, dst_p, es, ed, h, zrows, zs):
    k = pl.kernel(
        _sc_edge_body,
        out_type=[jax.ShapeDtypeStruct((NC, N_PAD, 128), jnp.float32),
                  jax.ShapeDtypeStruct((NW, N_PAD), jnp.float32)],
        mesh=_mesh,
        compiler_params=_cp,
        scratch_types=[
            pltpu.VMEM((NBLK, EB), jnp.int32),
            pltpu.VMEM((NBLK, 1, EB), jnp.int32),
            pltpu.VMEM((N_PAD,), jnp.float32),
            pltpu.VMEM((N_PAD,), jnp.float32),
            pltpu.VMEM((EB, DIM), jnp.float32),
            pltpu.VMEM((EB, DIM), jnp.float32),
            pltpu.VMEM((EB, UW), jnp.float32),
            pltpu.VMEM((EB, UW), jnp.float32),
            pltpu.VMEM((N_PAD,), jnp.float32),
            pltpu.VMEM_SHARED((N_PAD, UW), jnp.float32),
            pltpu.VMEM_SHARED((N_PAD, DIM), jnp.float32),
            pltpu.SemaphoreType.DMA,
            pltpu.SemaphoreType.DMA,
            pltpu.SemaphoreType.DMA,
            pltpu.SemaphoreType.DMA,
            pltpu.SemaphoreType.DMA,
        ],
    )
    return k(src_p, dst_p, es, ed, h, zrows, zs)


G_PAD = 256
PB = 80
NPB = 4


def _sc_pool_body(h_hbm, batch_hbm, zg_hbm, g_hbm, h_v, b_v, g_sh, sem):
    cid = lax.axis_index("c")
    tid = lax.axis_index("s")
    wid = tid * NC + cid
    rows = NPB * PB
    pltpu.async_copy(h_hbm.at[pl.ds(wid * rows, rows), pl.ds(0, DIM)], h_v, sem)
    pltpu.async_copy(batch_hbm.at[wid], b_v, sem)

    @pl.when(tid == 0)
    def _():
        pltpu.async_copy(zg_hbm, g_sh, sem)
        pltpu.make_async_copy(zg_hbm, g_sh, sem).wait()
    pltpu.make_async_copy(h_hbm.at[pl.ds(wid * rows, rows), pl.ds(0, DIM)], h_v, sem).wait()
    pltpu.make_async_copy(batch_hbm.at[wid], b_v, sem).wait()
    plsc.subcore_barrier()
    for k in range(NPB):
        pltpu.async_copy(
            h_v.at[pl.ds(k * PB, PB)], g_sh.at[b_v.at[k, 0]], sem, add=True)
    for k in range(NPB):
        pltpu.make_async_copy(
            h_v.at[pl.ds(k * PB, PB)], g_sh.at[b_v.at[k, 0]], sem).wait()
    plsc.subcore_barrier()

    @pl.when(tid == 0)
    def _():
        pltpu.sync_copy(g_sh, g_hbm.at[cid])


def _sc_pool(h3, batch_p, zg):
    k = pl.kernel(
        _sc_pool_body,
        out_type=jax.ShapeDtypeStruct((NC, G_PAD, DIM), jnp.float32),
        mesh=_mesh,
        compiler_params=_cp,
        scratch_types=[
            pltpu.VMEM((NPB * PB, DIM), jnp.float32),
            pltpu.VMEM((NPB, 1, PB), jnp.int32),
            pltpu.VMEM_SHARED((G_PAD, DIM), jnp.float32),
            pltpu.SemaphoreType.DMA,
        ],
    )
    return k(h3, batch_p, zg)


def _lay1_body(x_ref, w_ref, as_ref, ad_ref, h_ref, es_ref, ed_ref):
    hw = jnp.dot(x_ref[...], w_ref[...], preferred_element_type=jnp.float32)
    h_ref[:, pl.ds(0, DIM)] = hw
    es_ref[...] = hw @ as_ref[...]
    ed_ref[...] = hw @ ad_ref[...]


def _tc_layer1(x_pad, W1, att_src1, att_dst1):
    g = N_PAD // BR
    return pl.pallas_call(
        _lay1_body,
        grid=(g,),
        in_specs=[
            pl.BlockSpec((BR, F_IN), lambda i: (i, 0)),
            pl.BlockSpec((F_IN, DIM), lambda i: (0, 0)),
            pl.BlockSpec((DIM,), lambda i: (0,)),
            pl.BlockSpec((DIM,), lambda i: (0,)),
        ],
        out_specs=[
            pl.BlockSpec((BR, 128), lambda i: (i, 0)),
            pl.BlockSpec((BR,), lambda i: (i,)),
            pl.BlockSpec((BR,), lambda i: (i,)),
        ],
        out_shape=[
            jax.ShapeDtypeStruct((N_PAD, 128), jnp.float32),
            jax.ShapeDtypeStruct((N_PAD,), jnp.float32),
            jax.ShapeDtypeStruct((N_PAD,), jnp.float32),
        ],
    )(x_pad, W1, att_src1, att_dst1)


def _combine(u_ref, s_ref, b_ref):
    U = u_ref[0, :, :DIM] + u_ref[1, :, :DIM]
    den = jnp.sum(s_ref[...], axis=0)[:, None]
    agg = U / (den + 1e-16) + b_ref[...]
    nrm = jnp.sqrt(jnp.sum(agg * agg, axis=1, keepdims=True))
    return jax.nn.relu(agg / jnp.maximum(nrm, 1e-12))


def _mid_body(u_ref, s_ref, b_ref, w_ref, as_ref, ad_ref, h_ref, es_ref, ed_ref):
    hn = _combine(u_ref, s_ref, b_ref)
    hw = jnp.dot(hn, w_ref[...], preferred_element_type=jnp.float32)
    h_ref[:, pl.ds(0, DIM)] = hw
    es_ref[...] = hw @ as_ref[...]
    ed_ref[...] = hw @ ad_ref[...]


def _tc_mid(u, s, b_prev, W, att_src, att_dst):
    g = N_PAD // BR
    return pl.pallas_call(
        _mid_body,
        grid=(g,),
        in_specs=[
            pl.BlockSpec((NC, BR, 128), lambda i: (0, i, 0)),
            pl.BlockSpec((NW, BR), lambda i: (0, i)),
            pl.BlockSpec((DIM,), lambda i: (0,)),
            pl.BlockSpec((DIM, DIM), lambda i: (0, 0)),
            pl.BlockSpec((DIM,), lambda i: (0,)),
            pl.BlockSpec((DIM,), lambda i: (0,)),
        ],
        out_specs=[
            pl.BlockSpec((BR, 128), lambda i: (i, 0)),
            pl.BlockSpec((BR,), lambda i: (i,)),
            pl.BlockSpec((BR,), lambda i: (i,)),
        ],
        out_shape=[
            jax.ShapeDtypeStruct((N_PAD, 128), jnp.float32),
            jax.ShapeDtypeStruct((N_PAD,), jnp.float32),
            jax.ShapeDtypeStruct((N_PAD,), jnp.float32),
        ],
    )(u, s, b_prev, W, att_src, att_dst)


def _fin_body(u_ref, s_ref, b_ref, h_ref):
    h_ref[:, pl.ds(0, DIM)] = _combine(u_ref, s_ref, b_ref)


def _tc_final(u, s, b3):
    g = N_PAD // BR
    return pl.pallas_call(
        _fin_body,
        grid=(g,),
        in_specs=[
            pl.BlockSpec((NC, BR, 128), lambda i: (0, i, 0)),
            pl.BlockSpec((NW, BR), lambda i: (0, i)),
            pl.BlockSpec((DIM,), lambda i: (0,)),
        ],
        out_specs=pl.BlockSpec((BR, 128), lambda i: (i, 0)),
        out_shape=jax.ShapeDtypeStruct((N_PAD, 128), jnp.float32),
    )(u, s, b3)


def _head_body(g_ref, w1_ref, b1_ref, w2_ref, b2_ref, o_ref):
    gall = g_ref[...]
    g = gall[0, :NUM_GRAPHS] + gall[1, :NUM_GRAPHS]
    z = jax.nn.relu(g @ w1_ref[...] + b1_ref[...])
    z = z @ w2_ref[...] + b2_ref[...]
    o_ref[...] = jax.nn.log_softmax(z, axis=-1)


def _tc_head(g, fc1_W, fc1_b, fc2_W, fc2_b):
    return pl.pallas_call(
        _head_body,
        out_shape=jax.ShapeDtypeStruct((NUM_GRAPHS, NUM_CLASSES), jnp.float32),
    )(g, fc1_W, fc1_b, fc2_W, fc2_b)


def kernel(x, edge_index, batch, W1, att_src1, att_dst1, b1, W2, att_src2, att_dst2, b2, W3, att_src3, att_dst3, b3, fc1_W, fc1_b, fc2_W, fc2_b):
    loop = jnp.arange(N, dtype=edge_index.dtype)
    src = jnp.concatenate([edge_index[0], loop])
    dst = jnp.concatenate([edge_index[1], loop])
    pad = E_PAD - E_TOT
    src_p = jnp.pad(src, (0, pad)).reshape(NW, NBLK, EB)
    dst_p = jnp.pad(dst, (0, pad)).reshape(NW, NBLK, 1, EB)
    zrows = jnp.zeros((ROWS_PER_TEC, UW), jnp.float32)
    zs = jnp.zeros((N_PAD,), jnp.float32)
    x_pad = jnp.pad(x, ((0, N_PAD - N), (0, 0)))
    batch_p = jnp.pad(batch, (0, N_PAD - N),
                      constant_values=NUM_GRAPHS).reshape(NW, NPB, 1, PB)
    zg = jnp.zeros((G_PAD, DIM), jnp.float32)

    h, es, ed = _tc_layer1(x_pad, W1, att_src1, att_dst1)
    u, sden = _sc_edge(src_p, dst_p, es, ed, h, zrows, zs)
    h, es, ed = _tc_mid(u, sden, b1, W2, att_src2, att_dst2)
    u, sden = _sc_edge(src_p, dst_p, es, ed, h, zrows, zs)
    h, es, ed = _tc_mid(u, sden, b2, W3, att_src3, att_dst3)
    u, sden = _sc_edge(src_p, dst_p, es, ed, h, zrows, zs)
    h3 = _tc_final(u, sden, b3)

    g = _sc_pool(h3, batch_p, zg)
    return _tc_head(g, fc1_W, fc1_b, fc2_W, fc2_b)
